# Initial kernel scaffold; baseline (speedup 1.0000x reference)
import jax
import jax.numpy as jnp
from jax.experimental import pallas as pl


def kernel(point_set, feature_image, extrinsics, intrinsics):
    raise NotImplementedError("write your pallas kernel here")


# speedup vs baseline: 3.8958x; 3.8958x over previous
import functools

import jax
import jax.numpy as jnp
from jax import lax
from jax.experimental import pallas as pl
from jax.experimental.pallas import tpu as pltpu
from jax.experimental.pallas import tpu_sc as plsc

_B, _N, _K, _H, _W = 4, 16384, 128, 256, 256
_HW = _H * _W
_IMG_H = 256
_NC, _NS, _L = 2, 16, 16
_NW = _NC * _NS
_WPB = _NW // _B
_KPW = _K // _WPB
_SENT = _HW
_PAD = 272
_PCH = 2048
_OCH = 4096


def _sc_body(fi, coords, out, img, idxb, wxb, wyb, pxb, pyb, obuf):
    cid = lax.axis_index("c")
    sid = lax.axis_index("s")
    wid = sid * _NC + cid
    b = wid // _WPB
    kbase = (wid % _WPB) * _KPW

    zeros = jnp.zeros((_L,), jnp.float32)
    for g in range(_PAD // _L):
        img[pl.ds(_HW + g * _L, _L)] = zeros

    for c in range(_N // _PCH):
        base = c * _PCH
        pltpu.sync_copy(coords.at[2 * b + 0, pl.ds(base, _PCH)], pxb)
        pltpu.sync_copy(coords.at[2 * b + 1, pl.ds(base, _PCH)], pyb)

        def proj(g, carry):
            o = g * _L
            x = pxb[pl.ds(o, _L)]
            y = pyb[pl.ds(o, _L)]
            valid = (x >= 0.0) & (y >= 0.0) & (x < float(_IMG_H)) & (y < float(_IMG_H))
            xc = jnp.minimum(jnp.maximum(x, 1.0), float(_IMG_H - 2))
            yc = jnp.minimum(jnp.maximum(y, 1.0), float(_IMG_H - 2))
            xi = xc.astype(jnp.int32)
            yi = yc.astype(jnp.int32)
            wx = xc - xi.astype(jnp.float32)
            wy = yc - yi.astype(jnp.float32)
            idx = yi * _W + xi
            idxb[pl.ds(base + o, _L)] = jnp.where(valid, idx, _SENT)
            wxb[pl.ds(base + o, _L)] = jnp.where(valid, wx, 0.0)
            wyb[pl.ds(base + o, _L)] = jnp.where(valid, wy, 0.0)
            return carry

        lax.fori_loop(0, _PCH // _L, proj, 0)

    for t in range(_KPW):
        row = b * _K + kbase + t
        pltpu.sync_copy(fi.at[row], img.at[pl.ds(0, _HW)])
        for oc in range(_N // _OCH):
            ob = oc * _OCH

            def interp(g, carry):
                o = g * _L
                i0 = idxb[pl.ds(ob + o, _L)]
                wx = wxb[pl.ds(ob + o, _L)]
                wy = wyb[pl.ds(ob + o, _L)]
                f00 = plsc.load_gather(img, [i0])
                f01 = plsc.load_gather(img, [i0 + 1])
                f10 = plsc.load_gather(img, [i0 + _W])
                f11 = plsc.load_gather(img, [i0 + _W + 1])
                wx1 = 1.0 - wx
                a = f00 * wx1 + f01 * wx
                bb = f10 * wx1 + f11 * wx
                obuf[pl.ds(o, _L)] = a * (1.0 - wy) + bb * wy
                return carry

            lax.fori_loop(0, _OCH // _L, interp, 0)
            pltpu.sync_copy(obuf, out.at[row, pl.ds(ob, _OCH)])


@functools.partial(
    pl.kernel,
    out_type=jax.ShapeDtypeStruct((_B * _K, _N), jnp.float32),
    mesh=plsc.VectorSubcoreMesh(core_axis_name="c", subcore_axis_name="s"),
    compiler_params=pltpu.CompilerParams(needs_layout_passes=False),
    scratch_types=[
        pltpu.VMEM((_HW + _PAD,), jnp.float32),
        pltpu.VMEM((_N,), jnp.int32),
        pltpu.VMEM((_N,), jnp.float32),
        pltpu.VMEM((_N,), jnp.float32),
        pltpu.VMEM((_PCH,), jnp.float32),
        pltpu.VMEM((_PCH,), jnp.float32),
        pltpu.VMEM((_OCH,), jnp.float32),
    ],
)
def _sc_interp(fi, coords, out, img, idxb, wxb, wyb, pxb, pyb, obuf):
    _sc_body(fi, coords, out, img, idxb, wxb, wyb, pxb, pyb, obuf)


def kernel(point_set, feature_image, extrinsics, intrinsics):
    ps = jnp.concatenate([point_set, jnp.ones_like(point_set[:, :, 0:1])], axis=-1)
    ps_homog = jnp.transpose(ps, (0, 2, 1))
    cam_points = (jnp.linalg.inv(extrinsics).astype(jnp.float32) @ ps_homog)[:, :3]
    im_coords = intrinsics @ cam_points
    im_coords_homog = (im_coords / im_coords[:, -1:, :])[:, :2, :]

    fi = feature_image.reshape(_B * _K, _HW)
    coords = im_coords_homog.reshape(_B * 2, _N)
    out = _sc_interp(fi, coords)
    return out.reshape(_B, _K, _N)


# speedup vs baseline: 5.5406x; 1.4222x over previous
import functools

import jax
import jax.numpy as jnp
from jax import lax
from jax.experimental import pallas as pl
from jax.experimental.pallas import tpu as pltpu
from jax.experimental.pallas import tpu_sc as plsc

_B, _N, _K, _H, _W = 4, 16384, 128, 256, 256
_HW = _H * _W
_IMG_H = 256
_NC, _NS, _L = 2, 16, 16
_NW = _NC * _NS
_WPB = _NW // _B
_KPW = _K // _WPB
_SENT = _HW
_PAD = 272
_PCH = 2048
_OCH = 4096


def _sc_body(fi, coords, out, img, idxb, wxb, wyb, pxb, pyb, obufa, obufb, sema, semb):
    cid = lax.axis_index("c")
    sid = lax.axis_index("s")
    wid = sid * _NC + cid
    b = wid // _WPB
    kbase = (wid % _WPB) * _KPW

    zeros = jnp.zeros((_L,), jnp.float32)
    for g in range(_PAD // _L):
        img[pl.ds(_HW + g * _L, _L)] = zeros

    for c in range(_N // _PCH):
        base = c * _PCH
        pltpu.sync_copy(coords.at[2 * b + 0, pl.ds(base, _PCH)], pxb)
        pltpu.sync_copy(coords.at[2 * b + 1, pl.ds(base, _PCH)], pyb)

        @plsc.parallel_loop(0, _PCH // _L, unroll=4)
        def proj(g):
            o = g * _L
            x = pxb[pl.ds(o, _L)]
            y = pyb[pl.ds(o, _L)]
            valid = (x >= 0.0) & (y >= 0.0) & (x < float(_IMG_H)) & (y < float(_IMG_H))
            xc = jnp.minimum(jnp.maximum(x, 1.0), float(_IMG_H - 2))
            yc = jnp.minimum(jnp.maximum(y, 1.0), float(_IMG_H - 2))
            xi = xc.astype(jnp.int32)
            yi = yc.astype(jnp.int32)
            wx = xc - xi.astype(jnp.float32)
            wy = yc - yi.astype(jnp.float32)
            idx = yi * _W + xi
            idxb[pl.ds(base + o, _L)] = jnp.where(valid, idx, _SENT)
            wxb[pl.ds(base + o, _L)] = jnp.where(valid, wx, 0.0)
            wyb[pl.ds(base + o, _L)] = jnp.where(valid, wy, 0.0)

        del proj

    def chan(t, carry):
        row = b * _K + kbase + t
        pltpu.sync_copy(fi.at[row], img.at[pl.ds(0, _HW)])

        def compute_chunk(oc_static, obuf):
            ob = oc_static * _OCH

            @plsc.parallel_loop(0, _OCH // _L, unroll=4)
            def interp(g):
                o = g * _L
                i0 = idxb[pl.ds(ob + o, _L)]
                wx = wxb[pl.ds(ob + o, _L)]
                wy = wyb[pl.ds(ob + o, _L)]
                f00 = plsc.load_gather(img, [i0])
                f01 = plsc.load_gather(img, [i0 + 1])
                f10 = plsc.load_gather(img, [i0 + _W])
                f11 = plsc.load_gather(img, [i0 + _W + 1])
                wx1 = 1.0 - wx
                a = f00 * wx1 + f01 * wx
                bb = f10 * wx1 + f11 * wx
                obuf[pl.ds(o, _L)] = a * (1.0 - wy) + bb * wy

            del interp

        compute_chunk(0, obufa)
        cpa = pltpu.async_copy(obufa, out.at[row, pl.ds(0 * _OCH, _OCH)], sema)
        compute_chunk(1, obufb)
        cpb = pltpu.async_copy(obufb, out.at[row, pl.ds(1 * _OCH, _OCH)], semb)
        cpa.wait()
        compute_chunk(2, obufa)
        cpa = pltpu.async_copy(obufa, out.at[row, pl.ds(2 * _OCH, _OCH)], sema)
        cpb.wait()
        compute_chunk(3, obufb)
        cpb = pltpu.async_copy(obufb, out.at[row, pl.ds(3 * _OCH, _OCH)], semb)
        cpa.wait()
        cpb.wait()
        return carry

    lax.fori_loop(0, _KPW, chan, 0)


@functools.partial(
    pl.kernel,
    out_type=jax.ShapeDtypeStruct((_B * _K, _N), jnp.float32),
    mesh=plsc.VectorSubcoreMesh(core_axis_name="c", subcore_axis_name="s"),
    compiler_params=pltpu.CompilerParams(needs_layout_passes=False),
    scratch_types=[
        pltpu.VMEM((_HW + _PAD,), jnp.float32),
        pltpu.VMEM((_N,), jnp.int32),
        pltpu.VMEM((_N,), jnp.float32),
        pltpu.VMEM((_N,), jnp.float32),
        pltpu.VMEM((_PCH,), jnp.float32),
        pltpu.VMEM((_PCH,), jnp.float32),
        pltpu.VMEM((_OCH,), jnp.float32),
        pltpu.VMEM((_OCH,), jnp.float32),
        pltpu.SemaphoreType.DMA,
        pltpu.SemaphoreType.DMA,
    ],
)
def _sc_interp(fi, coords, out, img, idxb, wxb, wyb, pxb, pyb, obufa, obufb, sema, semb):
    _sc_body(fi, coords, out, img, idxb, wxb, wyb, pxb, pyb, obufa, obufb, sema, semb)


def kernel(point_set, feature_image, extrinsics, intrinsics):
    ps = jnp.concatenate([point_set, jnp.ones_like(point_set[:, :, 0:1])], axis=-1)
    ps_homog = jnp.transpose(ps, (0, 2, 1))
    cam_points = (jnp.linalg.inv(extrinsics).astype(jnp.float32) @ ps_homog)[:, :3]
    im_coords = intrinsics @ cam_points
    im_coords_homog = (im_coords / im_coords[:, -1:, :])[:, :2, :]

    fi = feature_image.reshape(_B * _K, _HW)
    coords = im_coords_homog.reshape(_B * 2, _N)
    out = _sc_interp(fi, coords)
    return out.reshape(_B, _K, _N)


# speedup vs baseline: 6.1788x; 1.1152x over previous
import functools

import jax
import jax.numpy as jnp
from jax import lax
from jax.experimental import pallas as pl
from jax.experimental.pallas import tpu as pltpu
from jax.experimental.pallas import tpu_sc as plsc

_B, _N, _K, _H, _W = 4, 16384, 128, 256, 256
_HW = _H * _W
_IMG_H = 256
_NC, _NS, _L = 2, 16, 16
_NW = _NC * _NS
_WPB = _NW // _B
_KPW = _K // _WPB
_SENT = _HW
_PAD = 272
_PCH = 2048
_OCH = 4096


def _sc_body(fi, coords, out, img, idxb, wxb, wyb, pxb, pyb, obufa, obufb, sema, semb):
    cid = lax.axis_index("c")
    sid = lax.axis_index("s")
    wid = sid * _NC + cid
    b = wid // _WPB
    kbase = (wid % _WPB) * _KPW

    zeros = jnp.zeros((_L,), jnp.float32)
    for r in (_H, _H + 1):
        for g in range(_W // _L):
            img[r, pl.ds(g * _L, _L)] = zeros

    for c in range(_N // _PCH):
        base = c * _PCH
        pltpu.sync_copy(coords.at[b, 0, pl.ds(base, _PCH)], pxb)
        pltpu.sync_copy(coords.at[b, 1, pl.ds(base, _PCH)], pyb)

        @plsc.parallel_loop(0, _PCH // _L, unroll=4)
        def proj(g):
            o = g * _L
            x = pxb[pl.ds(o, _L)]
            y = pyb[pl.ds(o, _L)]
            valid = (x >= 0.0) & (y >= 0.0) & (x < float(_IMG_H)) & (y < float(_IMG_H))
            xc = jnp.minimum(jnp.maximum(x, 1.0), float(_IMG_H - 2))
            yc = jnp.minimum(jnp.maximum(y, 1.0), float(_IMG_H - 2))
            xi = xc.astype(jnp.int32)
            yi = yc.astype(jnp.int32)
            wx = xc - xi.astype(jnp.float32)
            wy = yc - yi.astype(jnp.float32)
            idx = yi * _W + xi
            idxb[pl.ds(base + o, _L)] = jnp.where(valid, idx, _SENT)
            wxb[pl.ds(base + o, _L)] = jnp.where(valid, wx, 0.0)
            wyb[pl.ds(base + o, _L)] = jnp.where(valid, wy, 0.0)

        del proj

    def chan(t, carry):
        kk = kbase + t
        row = b * _K + kk
        pltpu.sync_copy(fi.at[b, kk], img.at[pl.ds(0, _H)])

        def compute_chunk(oc_static, obuf):
            ob = oc_static * _OCH

            @plsc.parallel_loop(0, _OCH // _L, unroll=4)
            def interp(g):
                o = g * _L
                i0 = idxb[pl.ds(ob + o, _L)]
                wx = wxb[pl.ds(ob + o, _L)]
                wy = wyb[pl.ds(ob + o, _L)]
                iy = lax.shift_right_logical(i0, 8)
                ix = lax.bitwise_and(i0, 255)
                iy1 = iy + 1
                ix1 = ix + 1
                f00 = plsc.load_gather(img, [iy, ix])
                f01 = plsc.load_gather(img, [iy, ix1])
                f10 = plsc.load_gather(img, [iy1, ix])
                f11 = plsc.load_gather(img, [iy1, ix1])
                wx1 = 1.0 - wx
                a = f00 * wx1 + f01 * wx
                bb = f10 * wx1 + f11 * wx
                obuf[pl.ds(o, _L)] = a * (1.0 - wy) + bb * wy

            del interp

        compute_chunk(0, obufa)
        cpa = pltpu.async_copy(obufa, out.at[row, pl.ds(0 * _OCH, _OCH)], sema)
        compute_chunk(1, obufb)
        cpb = pltpu.async_copy(obufb, out.at[row, pl.ds(1 * _OCH, _OCH)], semb)
        cpa.wait()
        compute_chunk(2, obufa)
        cpa = pltpu.async_copy(obufa, out.at[row, pl.ds(2 * _OCH, _OCH)], sema)
        cpb.wait()
        compute_chunk(3, obufb)
        cpb = pltpu.async_copy(obufb, out.at[row, pl.ds(3 * _OCH, _OCH)], semb)
        cpa.wait()
        cpb.wait()
        return carry

    lax.fori_loop(0, _KPW, chan, 0)


@functools.partial(
    pl.kernel,
    out_type=jax.ShapeDtypeStruct((_B * _K, _N), jnp.float32),
    mesh=plsc.VectorSubcoreMesh(core_axis_name="c", subcore_axis_name="s"),
    compiler_params=pltpu.CompilerParams(needs_layout_passes=False),
    scratch_types=[
        pltpu.VMEM((_H + 2, _W), jnp.float32),
        pltpu.VMEM((_N,), jnp.int32),
        pltpu.VMEM((_N,), jnp.float32),
        pltpu.VMEM((_N,), jnp.float32),
        pltpu.VMEM((_PCH,), jnp.float32),
        pltpu.VMEM((_PCH,), jnp.float32),
        pltpu.VMEM((_OCH,), jnp.float32),
        pltpu.VMEM((_OCH,), jnp.float32),
        pltpu.SemaphoreType.DMA,
        pltpu.SemaphoreType.DMA,
    ],
)
def _sc_interp(fi, coords, out, img, idxb, wxb, wyb, pxb, pyb, obufa, obufb, sema, semb):
    _sc_body(fi, coords, out, img, idxb, wxb, wyb, pxb, pyb, obufa, obufb, sema, semb)


def kernel(point_set, feature_image, extrinsics, intrinsics):
    ps = jnp.concatenate([point_set, jnp.ones_like(point_set[:, :, 0:1])], axis=-1)
    ps_homog = jnp.transpose(ps, (0, 2, 1))
    cam_points = (jnp.linalg.inv(extrinsics).astype(jnp.float32) @ ps_homog)[:, :3]
    im_coords = intrinsics @ cam_points
    im_coords_homog = (im_coords / im_coords[:, -1:, :])[:, :2, :]

    out = _sc_interp(feature_image, im_coords_homog)
    return out.reshape(_B, _K, _N)
